# Initial kernel scaffold; baseline (speedup 1.0000x reference)
import functools

import jax
import jax.numpy as jnp
from jax import lax
from jax.experimental import pallas as pl
from jax.experimental.pallas import tpu as pltpu
from jax.experimental.pallas import tpu_sc as plsc

B, L = 4096, 200
N = B * L
VOCAB = 100000
CHAR_LEN = 32
N_POS = 20
N_PRIV = 5
M = 1000
CTX_OBF_RATE = 0.1
PRI_OBF_RATE = 1.0

NC, NS, LANES = 2, 16, 16
NW = NC * NS
PER_W = N // NW
T = 2560
NCHUNK = PER_W // T


@functools.partial(
    pl.kernel,
    out_type=[
        jax.ShapeDtypeStruct((N,), jnp.int32),
        jax.ShapeDtypeStruct((N, CHAR_LEN), jnp.int32),
        jax.ShapeDtypeStruct((N,), jnp.int32),
        jax.ShapeDtypeStruct((N,), jnp.int32),
        jax.ShapeDtypeStruct((N,), jnp.int32),
    ],
    mesh=plsc.VectorSubcoreMesh(core_axis_name="c", subcore_axis_name="s"),
    scratch_types=[
        pltpu.VMEM((T,), jnp.int32),
        pltpu.VMEM((T,), jnp.int32),
        pltpu.VMEM((T,), jnp.float32),
        pltpu.VMEM((T,), jnp.float32),
        pltpu.VMEM((T,), jnp.int32),
        pltpu.VMEM((T,), jnp.int32),
        pltpu.VMEM((T,), jnp.int32),
        pltpu.VMEM((T,), jnp.int32),
        pltpu.VMEM((N_POS * M,), jnp.int32),
        pltpu.VMEM((T, CHAR_LEN), jnp.int32),
        pltpu.SemaphoreType.DMA,
    ],
)
def _sc_kernel(word_hbm, pos_hbm, ctx_hbm, pri_hbm, maski_hbm, cand_hbm,
               tgt_hbm, lut_hbm,
               obfw_hbm, char_hbm, obfm_hbm, prim_hbm, cpym_hbm,
               word_v, pos_v, ctx_v, pri_v, maski_v, addr_v, cidx_v, obfw_v,
               tgt_v, char_v, sem):
    wid = lax.axis_index("s") * NC + lax.axis_index("c")
    wbase = wid * PER_W
    pltpu.sync_copy(tgt_hbm, tgt_v)

    for c in range(NCHUNK):
        base = wbase + c * T
        pltpu.sync_copy(word_hbm.at[pl.ds(base, T)], word_v)
        pltpu.sync_copy(pos_hbm.at[pl.ds(base, T)], pos_v)
        pltpu.sync_copy(ctx_hbm.at[pl.ds(base, T)], ctx_v)
        pltpu.sync_copy(pri_hbm.at[pl.ds(base, T)], pri_v)
        pltpu.sync_copy(maski_hbm.at[pl.ds(base, T)], maski_v)

        def addr_body(i, _):
            sl = pl.ds(i * LANES, LANES)
            p = pos_v[sl]
            tok = base + i * LANES + lax.iota(jnp.int32, LANES)
            addr_v[sl] = p * N + tok
            return ()
        lax.fori_loop(0, T // LANES, addr_body, ())

        pltpu.async_copy(cand_hbm.at[addr_v], cidx_v, sem).wait()

        def sel_body(i, _):
            sl = pl.ds(i * LANES, LANES)
            p = pos_v[sl]
            w = word_v[sl]
            cx = ctx_v[sl]
            pr = pri_v[sl]
            mk = maski_v[sl]
            ci = cidx_v[sl]
            cand = plsc.load_gather(tgt_v, [p * M + ci])
            is_priv = p < N_PRIV
            take = jnp.where(is_priv, pr < PRI_OBF_RATE, cx < CTX_OBF_RATE)
            obfw = jnp.where(take, cand, w)
            obfw_v[sl] = obfw
            addr_v[sl] = take.astype(jnp.int32)
            cidx_v[sl] = is_priv.astype(jnp.int32)
            word_v[sl] = ((mk != 0) & (w == obfw)).astype(jnp.int32)
            return ()
        lax.fori_loop(0, T // LANES, sel_body, ())

        pltpu.async_copy(lut_hbm.at[obfw_v], char_v, sem).wait()

        pltpu.sync_copy(obfw_v, obfw_hbm.at[pl.ds(base, T)])
        pltpu.sync_copy(char_v, char_hbm.at[pl.ds(base, T)])
        pltpu.sync_copy(addr_v, obfm_hbm.at[pl.ds(base, T)])
        pltpu.sync_copy(cidx_v, prim_hbm.at[pl.ds(base, T)])
        pltpu.sync_copy(word_v, cpym_hbm.at[pl.ds(base, T)])


def kernel(inp_word, inp_char, inp_pos, inp_mask, tgtwords, lut,
           ctx_rand, pri_rand, cand_idx):
    word = inp_word.reshape(N)
    pos = inp_pos.reshape(N)
    ctx = ctx_rand.reshape(N)
    pri = pri_rand.reshape(N)
    maski = inp_mask.reshape(N).astype(jnp.int32)
    cand = cand_idx.reshape(N_POS * N)
    tgt = tgtwords.reshape(N_POS * M)
    obfw, char, obfm, prim, cpym = _sc_kernel(
        word, pos, ctx, pri, maski, cand, tgt, lut)
    return (obfw.reshape(B, L), inp_word,
            char.reshape(B, L, CHAR_LEN).astype(inp_char.dtype), inp_pos,
            obfm.reshape(B, L).astype(bool),
            prim.reshape(B, L).astype(bool),
            cpym.reshape(B, L).astype(bool))


# speedup vs baseline: 197.1842x; 197.1842x over previous
import functools

import jax
import jax.numpy as jnp
from jax import lax
from jax.experimental import pallas as pl
from jax.experimental.pallas import tpu as pltpu
from jax.experimental.pallas import tpu_sc as plsc

B, L = 4096, 200
N = B * L
VOCAB = 100000
CHAR_LEN = 32
N_POS = 20
N_PRIV = 5
M = 1000
CTX_OBF_RATE = 0.1
PRI_OBF_RATE = 1.0

NC, NS, LANES = 2, 16, 16
NW = NC * NS
PER_W = N // NW
T = 2560
NCHUNK = PER_W // T


@functools.partial(
    pl.kernel,
    out_type=[
        jax.ShapeDtypeStruct((N,), jnp.int32),
        jax.ShapeDtypeStruct((N, CHAR_LEN), jnp.int32),
        jax.ShapeDtypeStruct((N,), jnp.int32),
        jax.ShapeDtypeStruct((N,), jnp.int32),
        jax.ShapeDtypeStruct((N,), jnp.int32),
    ],
    mesh=plsc.VectorSubcoreMesh(core_axis_name="c", subcore_axis_name="s"),
    compiler_params=pltpu.CompilerParams(
        needs_layout_passes=False, use_tc_tiling_on_sc=False),
    scratch_types=[
        pltpu.VMEM((T,), jnp.int32),
        pltpu.VMEM((T,), jnp.int32),
        pltpu.VMEM((T,), jnp.float32),
        pltpu.VMEM((T,), jnp.float32),
        pltpu.VMEM((T,), jnp.int32),
        pltpu.VMEM((T,), jnp.int32),
        pltpu.VMEM((T,), jnp.int32),
        pltpu.VMEM((T,), jnp.int32),
        pltpu.VMEM((N_POS * M,), jnp.int32),
        pltpu.VMEM((T, CHAR_LEN), jnp.int32),
        pltpu.SemaphoreType.DMA,
    ],
)
def _sc_kernel(word_hbm, pos_hbm, ctx_hbm, pri_hbm, maski_hbm, cand_hbm,
               tgt_hbm, lut_hbm,
               obfw_hbm, char_hbm, obfm_hbm, prim_hbm, cpym_hbm,
               word_v, pos_v, ctx_v, pri_v, maski_v, addr_v, cidx_v, obfw_v,
               tgt_v, char_v, sem):
    wid = lax.axis_index("s") * NC + lax.axis_index("c")
    wbase = wid * PER_W
    pltpu.sync_copy(tgt_hbm, tgt_v)

    for c in range(NCHUNK):
        base = wbase + c * T
        pltpu.sync_copy(word_hbm.at[pl.ds(base, T)], word_v)
        pltpu.sync_copy(pos_hbm.at[pl.ds(base, T)], pos_v)
        pltpu.sync_copy(ctx_hbm.at[pl.ds(base, T)], ctx_v)
        pltpu.sync_copy(pri_hbm.at[pl.ds(base, T)], pri_v)
        pltpu.sync_copy(maski_hbm.at[pl.ds(base, T)], maski_v)

        def addr_body(i, _):
            sl = pl.ds(i * LANES, LANES)
            p = pos_v[sl]
            tok = base + i * LANES + lax.iota(jnp.int32, LANES)
            addr_v[sl] = p * N + tok
            return ()
        lax.fori_loop(0, T // LANES, addr_body, ())

        pltpu.async_copy(cand_hbm.at[addr_v], cidx_v, sem).wait()

        def sel_body(i, _):
            sl = pl.ds(i * LANES, LANES)
            p = pos_v[sl]
            w = word_v[sl]
            cx = ctx_v[sl]
            pr = pri_v[sl]
            mk = maski_v[sl]
            ci = cidx_v[sl]
            cand = plsc.load_gather(tgt_v, [p * M + ci])
            is_priv = p < N_PRIV
            take = jnp.where(is_priv, pr < PRI_OBF_RATE, cx < CTX_OBF_RATE)
            obfw = jnp.where(take, cand, w)
            obfw_v[sl] = obfw
            addr_v[sl] = take.astype(jnp.int32)
            cidx_v[sl] = is_priv.astype(jnp.int32)
            word_v[sl] = ((mk != 0) & (w == obfw)).astype(jnp.int32)
            return ()
        lax.fori_loop(0, T // LANES, sel_body, ())

        pltpu.async_copy(lut_hbm.at[obfw_v], char_v, sem).wait()

        pltpu.sync_copy(obfw_v, obfw_hbm.at[pl.ds(base, T)])
        pltpu.sync_copy(char_v, char_hbm.at[pl.ds(base, T)])
        pltpu.sync_copy(addr_v, obfm_hbm.at[pl.ds(base, T)])
        pltpu.sync_copy(cidx_v, prim_hbm.at[pl.ds(base, T)])
        pltpu.sync_copy(word_v, cpym_hbm.at[pl.ds(base, T)])


def kernel(inp_word, inp_char, inp_pos, inp_mask, tgtwords, lut,
           ctx_rand, pri_rand, cand_idx):
    word = inp_word.reshape(N)
    pos = inp_pos.reshape(N)
    ctx = ctx_rand.reshape(N)
    pri = pri_rand.reshape(N)
    maski = inp_mask.reshape(N).astype(jnp.int32)
    cand = cand_idx.reshape(N_POS * N)
    tgt = tgtwords.reshape(N_POS * M)
    obfw, char, obfm, prim, cpym = _sc_kernel(
        word, pos, ctx, pri, maski, cand, tgt, lut)
    return (obfw.reshape(B, L), inp_word,
            char.reshape(B, L, CHAR_LEN).astype(inp_char.dtype), inp_pos,
            obfm.reshape(B, L).astype(bool),
            prim.reshape(B, L).astype(bool),
            cpym.reshape(B, L).astype(bool))


# speedup vs baseline: 207.1488x; 1.0505x over previous
import functools

import jax
import jax.numpy as jnp
from jax import lax
from jax.experimental import pallas as pl
from jax.experimental.pallas import tpu as pltpu
from jax.experimental.pallas import tpu_sc as plsc

B, L = 4096, 200
N = B * L
VOCAB = 100000
CHAR_LEN = 32
N_POS = 20
N_PRIV = 5
M = 1000
CTX_OBF_RATE = 0.1
PRI_OBF_RATE = 1.0

NC, NS, LANES = 2, 16, 16
NW = NC * NS
PER_W = N // NW
T = 1280
NCHUNK = PER_W // T

_i32 = jnp.int32


@functools.partial(
    pl.kernel,
    out_type=[
        jax.ShapeDtypeStruct((N,), jnp.int32),
        jax.ShapeDtypeStruct((N, CHAR_LEN), jnp.int32),
        jax.ShapeDtypeStruct((N,), jnp.int32),
    ],
    mesh=plsc.VectorSubcoreMesh(core_axis_name="c", subcore_axis_name="s"),
    compiler_params=pltpu.CompilerParams(
        needs_layout_passes=False, use_tc_tiling_on_sc=False),
    scratch_types=[
        [pltpu.VMEM((T,), _i32)] * 2,
        [pltpu.VMEM((T,), _i32)] * 2,
        [pltpu.VMEM((T,), jnp.float32)] * 2,
        [pltpu.VMEM((T,), jnp.float32)] * 2,
        [pltpu.VMEM((T,), _i32)] * 2,
        [pltpu.VMEM((T,), _i32)] * 2,
        [pltpu.VMEM((T,), _i32)] * 2,
        [pltpu.VMEM((T,), _i32)] * 2,
        [pltpu.VMEM((T, CHAR_LEN), _i32)] * 2,
        pltpu.VMEM((N_POS * M,), _i32),
        [pltpu.SemaphoreType.DMA] * 2,
        [pltpu.SemaphoreType.DMA] * 2,
        [pltpu.SemaphoreType.DMA] * 2,
        pltpu.SemaphoreType.DMA,
    ],
)
def _sc_kernel(word_hbm, pos_hbm, ctx_hbm, pri_hbm, maski_hbm, cand_hbm,
               tgt_hbm, lut_hbm,
               obfw_hbm, char_hbm, flags_hbm,
               word_v, pos_v, ctx_v, pri_v, maski_v, addr_v, cidx_v, obfw_v,
               char_v, tgt_v, sem_in, sem_lut, sem_out, sem_cand):
    wid = lax.axis_index("s") * NC + lax.axis_index("c")
    wbase = wid * PER_W
    pltpu.sync_copy(tgt_hbm, tgt_v)

    def fire_inputs(c):
        p = c % 2
        base = wbase + c * T
        s = sem_in[p]
        return [
            pltpu.async_copy(word_hbm.at[pl.ds(base, T)], word_v[p], s),
            pltpu.async_copy(pos_hbm.at[pl.ds(base, T)], pos_v[p], s),
            pltpu.async_copy(ctx_hbm.at[pl.ds(base, T)], ctx_v[p], s),
            pltpu.async_copy(pri_hbm.at[pl.ds(base, T)], pri_v[p], s),
            pltpu.async_copy(maski_hbm.at[pl.ds(base, T)], maski_v[p], s),
        ]

    def fire_outputs(c):
        p = c % 2
        base = wbase + c * T
        s = sem_out[p]
        return [
            pltpu.async_copy(obfw_v[p], obfw_hbm.at[pl.ds(base, T)], s),
            pltpu.async_copy(char_v[p], char_hbm.at[pl.ds(base, T)], s),
            pltpu.async_copy(addr_v[p], flags_hbm.at[pl.ds(base, T)], s),
        ]

    in_h = {0: fire_inputs(0)}
    lut_h = {}
    out_h = {}

    for c in range(NCHUNK):
        p = c % 2
        base = wbase + c * T
        if c >= 2:
            for h in out_h.pop(c - 2):
                h.wait()
        for h in in_h.pop(c):
            h.wait()

        def addr_body(i, _):
            sl = pl.ds(i * LANES, LANES)
            pd = pos_v[p][sl]
            tok = base + i * LANES + lax.iota(_i32, LANES)
            addr_v[p][sl] = pd * N + tok
            return ()
        lax.fori_loop(0, T // LANES, addr_body, ())

        pltpu.async_copy(cand_hbm.at[addr_v[p]], cidx_v[p], sem_cand).wait()

        def sel_body(i, _):
            sl = pl.ds(i * LANES, LANES)
            pd = pos_v[p][sl]
            w = word_v[p][sl]
            cx = ctx_v[p][sl]
            pr = pri_v[p][sl]
            mk = maski_v[p][sl]
            ci = cidx_v[p][sl]
            cand = plsc.load_gather(tgt_v, [pd * M + ci])
            is_priv = pd < N_PRIV
            take = jnp.where(is_priv, pr < PRI_OBF_RATE, cx < CTX_OBF_RATE)
            obfw = jnp.where(take, cand, w)
            obfw_v[p][sl] = obfw
            cpy = (mk != 0) & (w == obfw)
            addr_v[p][sl] = (take.astype(_i32)
                             | (is_priv.astype(_i32) << 1)
                             | (cpy.astype(_i32) << 2))
            return ()
        lax.fori_loop(0, T // LANES, sel_body, ())

        if c >= 1:
            for h in lut_h.pop(c - 1):
                h.wait()
            out_h[c - 1] = fire_outputs(c - 1)

        lut_h[c] = [pltpu.async_copy(lut_hbm.at[obfw_v[p]], char_v[p],
                                     sem_lut[p])]
        if c + 1 < NCHUNK:
            in_h[c + 1] = fire_inputs(c + 1)

    last = NCHUNK - 1
    for h in lut_h.pop(last):
        h.wait()
    out_h[last] = fire_outputs(last)
    for c in (last - 1, last):
        for h in out_h.pop(c):
            h.wait()


def kernel(inp_word, inp_char, inp_pos, inp_mask, tgtwords, lut,
           ctx_rand, pri_rand, cand_idx):
    word = inp_word.reshape(N)
    pos = inp_pos.reshape(N)
    ctx = ctx_rand.reshape(N)
    pri = pri_rand.reshape(N)
    maski = inp_mask.reshape(N).astype(jnp.int32)
    cand = cand_idx.reshape(N_POS * N)
    tgt = tgtwords.reshape(N_POS * M)
    obfw, char, flags = _sc_kernel(word, pos, ctx, pri, maski, cand, tgt, lut)
    flags = flags.reshape(B, L)
    return (obfw.reshape(B, L), inp_word,
            char.reshape(B, L, CHAR_LEN).astype(inp_char.dtype), inp_pos,
            (flags & 1).astype(bool),
            (flags & 2).astype(bool),
            (flags & 4).astype(bool))


# speedup vs baseline: 261.6165x; 1.2629x over previous
import functools

import jax
import jax.numpy as jnp
from jax import lax
from jax.experimental import pallas as pl
from jax.experimental.pallas import tpu as pltpu
from jax.experimental.pallas import tpu_sc as plsc

B, L = 4096, 200
N = B * L
VOCAB = 100000
CHAR_LEN = 32
N_POS = 20
N_PRIV = 5
M = 1000
CTX_OBF_RATE = 0.1
PRI_OBF_RATE = 1.0

NC, NS, LANES = 2, 16, 16
NW = NC * NS

TR, TC_ = 25, 32
TILES = TR * TC_
TPW = TILES // NW
TILE_TOK = 1024

PER_W = N // NW
TB = 1600
NCB = PER_W // TB

_i32 = jnp.int32


@functools.partial(
    pl.kernel,
    out_type=[
        jax.ShapeDtypeStruct((200, 4096), jnp.int32),
        jax.ShapeDtypeStruct((N,), jnp.int32),
        jax.ShapeDtypeStruct((200, 4096), jnp.int32),
    ],
    mesh=plsc.VectorSubcoreMesh(core_axis_name="c", subcore_axis_name="s"),
    compiler_params=pltpu.CompilerParams(
        needs_layout_passes=False, use_tc_tiling_on_sc=True),
    scratch_types=[
        [pltpu.VMEM((8, 128), _i32)] * 2,
        [pltpu.VMEM((8, 128), _i32)] * 2,
        [pltpu.VMEM((8, 128), jnp.float32)] * 2,
        [pltpu.VMEM((8, 128), jnp.float32)] * 2,
        [pltpu.VMEM((8, 128), _i32)] * 2,
        [pltpu.VMEM((N_POS, 8, 128), _i32)] * 2,
        [pltpu.VMEM((8, 128), _i32)] * 2,
        [pltpu.VMEM((TILE_TOK,), _i32)] * 2,
        [pltpu.VMEM((8, 128), _i32)] * 2,
        pltpu.VMEM((N_POS * M,), _i32),
        [pltpu.SemaphoreType.DMA] * 2,
        [pltpu.SemaphoreType.DMA] * 2,
    ],
)
def _stage_a(word_hbm, pos_hbm, ctx_hbm, pri_hbm, maski_hbm, cand_hbm,
             tgt_hbm,
             obfw2_hbm, obfw1_hbm, flags_hbm,
             word_v, pos_v, ctx_v, pri_v, maski_v, cand_v, obfw_v, obfwf_v,
             flags_v, tgt_v, sem_in, sem_out):
    wid = lax.axis_index("s") * NC + lax.axis_index("c")
    pltpu.sync_copy(tgt_hbm, tgt_v)

    def tile_rc(k):
        tid = wid * TPW + k
        return (tid // TC_) * 8, (tid % TC_) * 128

    def fire_inputs(k):
        p = k % 2
        r0, c0 = tile_rc(k)
        s = sem_in[p]
        rs, cs = pl.ds(r0, 8), pl.ds(c0, 128)
        hs = [
            pltpu.async_copy(word_hbm.at[rs, cs], word_v[p], s),
            pltpu.async_copy(pos_hbm.at[rs, cs], pos_v[p], s),
            pltpu.async_copy(ctx_hbm.at[rs, cs], ctx_v[p], s),
            pltpu.async_copy(pri_hbm.at[rs, cs], pri_v[p], s),
            pltpu.async_copy(maski_hbm.at[rs, cs], maski_v[p], s),
        ]
        for q in range(N_POS):
            hs.append(pltpu.async_copy(cand_hbm.at[q, rs, cs],
                                       cand_v[p].at[q], s))
        return hs

    def fire_outputs(k):
        p = k % 2
        r0, c0 = tile_rc(k)
        s = sem_out[p]
        tid = wid * TPW + k
        return [
            pltpu.async_copy(obfw_v[p], obfw2_hbm.at[pl.ds(r0, 8), pl.ds(c0, 128)], s),
            pltpu.async_copy(obfwf_v[p], obfw1_hbm.at[pl.ds(tid * TILE_TOK, TILE_TOK)], s),
            pltpu.async_copy(flags_v[p], flags_hbm.at[pl.ds(r0, 8), pl.ds(c0, 128)], s),
        ]

    in_h = {0: fire_inputs(0)}
    out_h = {}
    for k in range(TPW):
        p = k % 2
        if k >= 2:
            for h in out_h.pop(k - 2):
                h.wait()
        for h in in_h.pop(k):
            h.wait()

        def body(i, _):
            r = i // 8
            sl = pl.ds((i % 8) * LANES, LANES)
            cv = (i % 8) * LANES + lax.iota(_i32, LANES)
            rv = jnp.full((LANES,), r, _i32)
            pd = pos_v[p][r, sl]
            w = word_v[p][r, sl]
            cx = ctx_v[p][r, sl]
            pr = pri_v[p][r, sl]
            mk = maski_v[p][r, sl]
            ci = plsc.load_gather(cand_v[p], [pd, rv, cv])
            cand = plsc.load_gather(tgt_v, [pd * M + ci])
            is_priv = pd < N_PRIV
            take = jnp.where(is_priv, pr < PRI_OBF_RATE, cx < CTX_OBF_RATE)
            obfw = jnp.where(take, cand, w)
            obfw_v[p][r, sl] = obfw
            obfwf_v[p][pl.ds(i * LANES, LANES)] = obfw
            cpy = (mk != 0) & (w == obfw)
            flags_v[p][r, sl] = (take.astype(_i32)
                                 | (is_priv.astype(_i32) << 1)
                                 | (cpy.astype(_i32) << 2))
            return ()
        lax.fori_loop(0, 64, body, ())

        out_h[k] = fire_outputs(k)
        if k + 1 < TPW:
            in_h[k + 1] = fire_inputs(k + 1)
    for k in (TPW - 2, TPW - 1):
        for h in out_h.pop(k):
            h.wait()


@functools.partial(
    pl.kernel,
    out_type=[
        jax.ShapeDtypeStruct((N, CHAR_LEN), jnp.int32),
    ],
    mesh=plsc.VectorSubcoreMesh(core_axis_name="c", subcore_axis_name="s"),
    compiler_params=pltpu.CompilerParams(
        needs_layout_passes=False, use_tc_tiling_on_sc=False),
    scratch_types=[
        [pltpu.VMEM((TB,), _i32)] * 2,
        [pltpu.VMEM((TB,), _i32)] * 2,
        [pltpu.VMEM((TB, CHAR_LEN), _i32)] * 2,
        [pltpu.SemaphoreType.DMA] * 2,
        [pltpu.SemaphoreType.DMA] * 2,
        [pltpu.SemaphoreType.DMA] * 2,
    ],
)
def _stage_b(obfw_hbm, lut_hbm,
             char_hbm,
             obfw_v, ridx_v, char_v, sem_in, sem_g, sem_s):
    wid = lax.axis_index("s") * NC + lax.axis_index("c")
    wbase = wid * PER_W

    def fire_input(c):
        p = c % 2
        return [pltpu.async_copy(obfw_hbm.at[pl.ds(wbase + c * TB, TB)],
                                 obfw_v[p], sem_in[p])]

    in_h = {0: fire_input(0)}
    sc_h = {}
    for c in range(NCB):
        p = c % 2
        if c >= 2:
            for h in sc_h.pop(c - 2):
                h.wait()
        for h in in_h.pop(c):
            h.wait()
        if c + 1 < NCB:
            in_h[c + 1] = fire_input(c + 1)

        q0 = wbase + c * TB

        def body(i, _):
            sl = pl.ds(i * LANES, LANES)
            q = q0 + i * LANES + lax.iota(_i32, LANES)
            tid = q >> 10
            bb = ((tid & 31) << 7) | (q & 127)
            ll = ((tid >> 5) << 3) | ((q >> 7) & 7)
            ridx_v[p][sl] = bb * L + ll
            return ()
        lax.fori_loop(0, TB // LANES, body, ())

        pltpu.async_copy(lut_hbm.at[obfw_v[p]], char_v[p], sem_g[p]).wait()
        sc_h[c] = [pltpu.async_copy(char_v[p], char_hbm.at[ridx_v[p]],
                                    sem_s[p])]
    for c in (NCB - 2, NCB - 1):
        for h in sc_h.pop(c):
            h.wait()


def kernel(inp_word, inp_char, inp_pos, inp_mask, tgtwords, lut,
           ctx_rand, pri_rand, cand_idx):
    wordT = inp_word.T
    posT = inp_pos.T
    ctxT = ctx_rand.T
    priT = pri_rand.T
    maskTi = inp_mask.T.astype(jnp.int32)
    candT = jnp.transpose(cand_idx, (0, 2, 1))
    tgt = tgtwords.reshape(N_POS * M)
    obfw2, obfw1, flags2 = _stage_a(wordT, posT, ctxT, priT, maskTi, candT, tgt)
    (char,) = _stage_b(obfw1, lut)
    flags = flags2.T
    return (obfw2.T, inp_word,
            char.reshape(B, L, CHAR_LEN), inp_pos,
            (flags & 1).astype(bool),
            (flags & 2).astype(bool),
            (flags & 4).astype(bool))
